# baseline (speedup 1.0000x reference)
import jax
import jax.numpy as jnp
from jax import lax
from jax.experimental import pallas as pl
from jax.experimental.pallas import tpu as pltpu
from jax.experimental.pallas import tpu_sc as plsc

D = 128
DW = D // 2
L = 16
NC = 2
NS = 16
NW = NC * NS
CE = 40
NB = 3
SC_G = 50
SC_E = SC_G * CE


def _make_sc_call(N, E):
    epw = E // NW
    gpw = epw // CE
    scn = gpw // SC_G
    assert epw * NW == E and gpw * CE == epw and scn * SC_G == gpw
    assert NB < SC_G

    mesh = plsc.VectorSubcoreMesh(
        core_axis_name="c", subcore_axis_name="s", num_cores=NC, num_subcores=NS
    )

    def sc_call(rp_hbm, a0_hbm, a1_hbm, out_hbm,
                table, idx0_v, idx1_v, rows0, rows1, sums, gsem, osem, isem):
        sid = lax.axis_index("s")
        wid = sid * NC + lax.axis_index("c")

        @pl.when(sid == 0)
        def _():
            pltpu.sync_copy(rp_hbm, table)

        eb = wid * epw
        pltpu.sync_copy(a0_hbm.at[pl.ds(eb, SC_E)], idx0_v.at[pl.ds(0, SC_E)])
        pltpu.sync_copy(a1_hbm.at[pl.ds(eb, SC_E)], idx1_v.at[pl.ds(0, SC_E)])
        plsc.subcore_barrier()

        def start_gathers(g):
            b = lax.rem(g, NB)
            off = lax.rem(g // SC_G, 2) * SC_E + lax.rem(g, SC_G) * CE
            pltpu.async_copy(
                table.at[idx0_v.at[pl.ds(off, CE)]], rows0.at[b], gsem.at[b]
            )
            pltpu.async_copy(
                table.at[idx1_v.at[pl.ds(off, CE)]], rows1.at[b], gsem.at[b]
            )

        for g in range(NB):
            start_gathers(g)

        mask_hi = jnp.full((L,), -65536, jnp.int32)
        sh16 = jnp.full((L,), 16, jnp.int32)

        def body(g, carry):
            b = lax.rem(g, NB)
            gmod = lax.rem(g, SC_G)

            @pl.when((gmod == 0) & (g < SC_G * (scn - 1)))
            def _():
                mm = g // SC_G + 1
                half = lax.rem(mm, 2)
                pltpu.async_copy(
                    a0_hbm.at[pl.ds(eb + mm * SC_E, SC_E)],
                    idx0_v.at[pl.ds(half * SC_E, SC_E)],
                    isem,
                )
                pltpu.async_copy(
                    a1_hbm.at[pl.ds(eb + mm * SC_E, SC_E)],
                    idx1_v.at[pl.ds(half * SC_E, SC_E)],
                    isem,
                )

            @pl.when((gmod == SC_G - NB) & (g < SC_G * (scn - 1)))
            def _():
                pltpu.make_async_copy(
                    a0_hbm.at[pl.ds(0, SC_E)], idx0_v.at[pl.ds(0, SC_E)], isem
                ).wait()
                pltpu.make_async_copy(
                    a1_hbm.at[pl.ds(0, SC_E)], idx1_v.at[pl.ds(0, SC_E)], isem
                ).wait()

            for _ in range(2):
                pltpu.make_async_copy(
                    table.at[idx0_v.at[pl.ds(0, CE)]], rows0.at[b], gsem.at[b]
                ).wait()

            @pl.when(g >= NB)
            def _():
                pltpu.make_async_copy(
                    sums.at[b], out_hbm.at[pl.ds(0, CE)], osem.at[b]
                ).wait()

            @plsc.parallel_loop(0, CE, unroll=8)
            def _(i):
                for j in range(DW // L):
                    sl = pl.ds(j * L, L)
                    v0 = rows0[b, i, sl]
                    v1 = rows1[b, i, sl]
                    lo = (lax.bitcast_convert_type(lax.shift_left(v0, sh16), jnp.float32)
                          + lax.bitcast_convert_type(lax.shift_left(v1, sh16), jnp.float32))
                    hi = (lax.bitcast_convert_type(lax.bitwise_and(v0, mask_hi), jnp.float32)
                          + lax.bitcast_convert_type(lax.bitwise_and(v1, mask_hi), jnp.float32))
                    sums[b, i, pl.ds(j * L, L)] = lo
                    sums[b, i, pl.ds(DW + j * L, L)] = hi

            @pl.when(g + NB < gpw)
            def _():
                start_gathers(g + NB)

            pltpu.async_copy(
                sums.at[b], out_hbm.at[pl.ds(eb + g * CE, CE)], osem.at[b]
            )
            return carry

        lax.fori_loop(0, gpw, body, 0)
        for b in range(NB):
            pltpu.make_async_copy(sums.at[b], out_hbm.at[pl.ds(0, CE)], osem.at[b]).wait()

    return pl.kernel(
        sc_call,
        mesh=mesh,
        compiler_params=pltpu.CompilerParams(use_tc_tiling_on_sc=False),
        out_type=jax.ShapeDtypeStruct((E, D), jnp.float32),
        scratch_types=[
            pltpu.VMEM_SHARED((N, DW), jnp.int32),
            pltpu.VMEM((2 * SC_E,), jnp.int32),
            pltpu.VMEM((2 * SC_E,), jnp.int32),
            pltpu.VMEM((NB, CE, DW), jnp.int32),
            pltpu.VMEM((NB, CE, DW), jnp.int32),
            pltpu.VMEM((NB, CE, D), jnp.float32),
            pltpu.SemaphoreType.DMA((NB,)),
            pltpu.SemaphoreType.DMA((NB,)),
            pltpu.SemaphoreType.DMA,
        ],
    )


def kernel(r, e, a):
    del e
    E = a.shape[0]
    a = a.astype(jnp.int32)
    rb = r.astype(jnp.bfloat16)
    half = r.shape[1] // 2
    lo = jax.lax.bitcast_convert_type(rb[:, :half], jnp.uint16).astype(jnp.uint32)
    hi = jax.lax.bitcast_convert_type(rb[:, half:], jnp.uint16).astype(jnp.uint32)
    rp = jax.lax.bitcast_convert_type(lo | (hi << 16), jnp.int32)
    return _make_sc_call(r.shape[0], E)(rp, a[:, 0], a[:, 1])


# speedup vs baseline: 1.0069x; 1.0069x over previous
import jax
import jax.numpy as jnp
from jax import lax
from jax.experimental import pallas as pl
from jax.experimental.pallas import tpu as pltpu
from jax.experimental.pallas import tpu_sc as plsc

D = 128
DW = D // 2
L = 16
NC = 2
NS = 16
NW = NC * NS
CE = 40
NB = 3
SC_G = 50
SC_E = SC_G * CE


def _make_sc_call(N, E):
    epw = E // NW
    gpw = epw // CE
    scn = gpw // SC_G
    assert epw * NW == E and gpw * CE == epw and scn * SC_G == gpw
    assert NB < SC_G

    mesh = plsc.VectorSubcoreMesh(
        core_axis_name="c", subcore_axis_name="s", num_cores=NC, num_subcores=NS
    )

    def sc_call(rp_hbm, a0_hbm, a1_hbm, out_hbm,
                table, idx0_v, idx1_v, rows0, rows1, sums, gsem, osem, isem):
        sid = lax.axis_index("s")
        wid = sid * NC + lax.axis_index("c")

        @pl.when(sid == 0)
        def _():
            pltpu.sync_copy(rp_hbm, table)

        eb = wid * epw
        pltpu.sync_copy(a0_hbm.at[pl.ds(eb, SC_E)], idx0_v.at[pl.ds(0, SC_E)])
        pltpu.sync_copy(a1_hbm.at[pl.ds(eb, SC_E)], idx1_v.at[pl.ds(0, SC_E)])
        plsc.subcore_barrier()

        def start_gathers(g):
            b = lax.rem(g, NB)
            off = lax.rem(g // SC_G, 2) * SC_E + lax.rem(g, SC_G) * CE
            pltpu.async_copy(
                table.at[idx0_v.at[pl.ds(off, CE)]], rows0.at[b], gsem.at[b]
            )
            pltpu.async_copy(
                table.at[idx1_v.at[pl.ds(off, CE)]], rows1.at[b], gsem.at[b]
            )

        for g in range(NB):
            start_gathers(g)

        mask_hi = jnp.full((L,), -65536, jnp.int32)
        sh16 = jnp.full((L,), 16, jnp.int32)

        def body(g, carry):
            b = lax.rem(g, NB)
            gmod = lax.rem(g, SC_G)

            @pl.when((gmod == 0) & (g < SC_G * (scn - 1)))
            def _():
                mm = g // SC_G + 1
                half = lax.rem(mm, 2)
                pltpu.async_copy(
                    a0_hbm.at[pl.ds(eb + mm * SC_E, SC_E)],
                    idx0_v.at[pl.ds(half * SC_E, SC_E)],
                    isem,
                )
                pltpu.async_copy(
                    a1_hbm.at[pl.ds(eb + mm * SC_E, SC_E)],
                    idx1_v.at[pl.ds(half * SC_E, SC_E)],
                    isem,
                )

            @pl.when((gmod == SC_G - NB) & (g < SC_G * (scn - 1)))
            def _():
                pltpu.make_async_copy(
                    a0_hbm.at[pl.ds(0, SC_E)], idx0_v.at[pl.ds(0, SC_E)], isem
                ).wait()
                pltpu.make_async_copy(
                    a1_hbm.at[pl.ds(0, SC_E)], idx1_v.at[pl.ds(0, SC_E)], isem
                ).wait()

            for _ in range(2):
                pltpu.make_async_copy(
                    table.at[idx0_v.at[pl.ds(0, CE)]], rows0.at[b], gsem.at[b]
                ).wait()

            @pl.when(g >= NB)
            def _():
                pltpu.make_async_copy(
                    sums.at[b], out_hbm.at[pl.ds(0, CE)], osem.at[b]
                ).wait()

            @plsc.parallel_loop(0, CE, unroll=4)
            def _(i):
                for j in range(DW // L):
                    sl = pl.ds(j * L, L)
                    v0 = rows0[b, i, sl]
                    v1 = rows1[b, i, sl]
                    lo = (lax.bitcast_convert_type(lax.shift_left(v0, sh16), jnp.float32)
                          + lax.bitcast_convert_type(lax.shift_left(v1, sh16), jnp.float32))
                    hi = (lax.bitcast_convert_type(lax.bitwise_and(v0, mask_hi), jnp.float32)
                          + lax.bitcast_convert_type(lax.bitwise_and(v1, mask_hi), jnp.float32))
                    sums[b, i, pl.ds(j * L, L)] = lo
                    sums[b, i, pl.ds(DW + j * L, L)] = hi

            @pl.when(g + NB < gpw)
            def _():
                start_gathers(g + NB)

            pltpu.async_copy(
                sums.at[b], out_hbm.at[pl.ds(eb + g * CE, CE)], osem.at[b]
            )
            return carry

        lax.fori_loop(0, gpw, body, 0)
        for b in range(NB):
            pltpu.make_async_copy(sums.at[b], out_hbm.at[pl.ds(0, CE)], osem.at[b]).wait()

    return pl.kernel(
        sc_call,
        mesh=mesh,
        compiler_params=pltpu.CompilerParams(use_tc_tiling_on_sc=False),
        out_type=jax.ShapeDtypeStruct((E, D), jnp.float32),
        scratch_types=[
            pltpu.VMEM_SHARED((N, DW), jnp.int32),
            pltpu.VMEM((2 * SC_E,), jnp.int32),
            pltpu.VMEM((2 * SC_E,), jnp.int32),
            pltpu.VMEM((NB, CE, DW), jnp.int32),
            pltpu.VMEM((NB, CE, DW), jnp.int32),
            pltpu.VMEM((NB, CE, D), jnp.float32),
            pltpu.SemaphoreType.DMA((NB,)),
            pltpu.SemaphoreType.DMA((NB,)),
            pltpu.SemaphoreType.DMA,
        ],
    )


def kernel(r, e, a):
    del e
    E = a.shape[0]
    a = a.astype(jnp.int32)
    rb = r.astype(jnp.bfloat16)
    half = r.shape[1] // 2
    lo = jax.lax.bitcast_convert_type(rb[:, :half], jnp.uint16).astype(jnp.uint32)
    hi = jax.lax.bitcast_convert_type(rb[:, half:], jnp.uint16).astype(jnp.uint32)
    rp = jax.lax.bitcast_convert_type(lo | (hi << 16), jnp.int32)
    return _make_sc_call(r.shape[0], E)(rp, a[:, 0], a[:, 1])
